# Initial kernel scaffold; baseline (speedup 1.0000x reference)
import jax
import jax.numpy as jnp
from jax.experimental import pallas as pl


def kernel(user_ids, item_ids, user_emb, item_emb, W1, b1, W2, b2, W3, b3, W4, b4):
    raise NotImplementedError("write your pallas kernel here")


# speedup vs baseline: 1.1557x; 1.1557x over previous
import functools

import jax
import jax.numpy as jnp
from jax import lax
from jax.experimental import pallas as pl
from jax.experimental.pallas import tpu as pltpu
from jax.experimental.pallas import tpu_sc as plsc

B = 16384
D = 64
IDX_CHUNK = 128


def _sc_geometry():
    try:
        info = plsc.get_sparse_core_info()
        return info.num_cores, info.num_subcores
    except Exception:
        return 2, 16


@functools.cache
def _make_gather(NC, NS):
    NW = NC * NS
    bpw = B // NW
    nch = bpw // IDX_CHUNK
    mesh = plsc.VectorSubcoreMesh(core_axis_name="c", subcore_axis_name="s")

    @functools.partial(
        pl.kernel,
        out_type=(
            jax.ShapeDtypeStruct((B, D), jnp.float32),
            jax.ShapeDtypeStruct((B, D), jnp.float32),
        ),
        mesh=mesh,
        scratch_types=[
            pltpu.VMEM((nch, IDX_CHUNK), jnp.int32),
            pltpu.VMEM((nch, IDX_CHUNK), jnp.int32),
            pltpu.VMEM((bpw, D), jnp.float32),
            pltpu.VMEM((bpw, D), jnp.float32),
            pltpu.SemaphoreType.DMA,
        ],
        compiler_params=pltpu.CompilerParams(use_tc_tiling_on_sc=False),
    )
    def gather(uids_hbm, iids_hbm, utab_hbm, itab_hbm, u_out, v_out,
               uidx, iidx, urows, vrows, sem):
        wid = lax.axis_index("s") * NC + lax.axis_index("c")
        base = wid * bpw
        pltpu.sync_copy(uids_hbm.at[pl.ds(wid * nch, nch)], uidx)
        pltpu.sync_copy(iids_hbm.at[pl.ds(wid * nch, nch)], iidx)
        copies = []
        for j in range(nch):
            dst = pl.ds(j * IDX_CHUNK, IDX_CHUNK)
            copies.append(pltpu.async_copy(utab_hbm.at[uidx.at[j]], urows.at[dst], sem))
            copies.append(pltpu.async_copy(itab_hbm.at[iidx.at[j]], vrows.at[dst], sem))
        for cp in copies:
            cp.wait()
        pltpu.sync_copy(urows, u_out.at[pl.ds(base, bpw)])
        pltpu.sync_copy(vrows, v_out.at[pl.ds(base, bpw)])

    return gather


def _mlp_body(u_ref, v_ref, w1_ref, b1_ref, w2_ref, b2_ref, w3_ref, b3_ref,
              w4_ref, b4_ref, out_ref):
    f32 = jnp.float32
    x = (jnp.dot(u_ref[...], w1_ref[0:D, :], preferred_element_type=f32)
         + jnp.dot(v_ref[...], w1_ref[D:2 * D, :], preferred_element_type=f32)
         + b1_ref[...])
    x = jnp.maximum(x, 0.0)
    x = jnp.maximum(jnp.dot(x, w2_ref[...], preferred_element_type=f32) + b2_ref[...], 0.0)
    x = jnp.maximum(jnp.dot(x, w3_ref[...], preferred_element_type=f32) + b3_ref[...], 0.0)
    logit = jnp.sum(x * w4_ref[...], axis=1, keepdims=True) + b4_ref[...]
    out_ref[...] = jax.nn.sigmoid(logit)


def kernel(user_ids, item_ids, user_emb, item_emb, W1, b1, W2, b2, W3, b3, W4, b4):
    NC, NS = _sc_geometry()
    uids2 = user_ids.astype(jnp.int32).reshape(B // IDX_CHUNK, IDX_CHUNK)
    iids2 = item_ids.astype(jnp.int32).reshape(B // IDX_CHUNK, IDX_CHUNK)
    u, v = _make_gather(NC, NS)(uids2, iids2, user_emb, item_emb)

    BB = 2048
    grid = (B // BB,)
    full = lambda shape: pl.BlockSpec(shape, lambda i: (0, 0))
    out = pl.pallas_call(
        _mlp_body,
        grid=grid,
        in_specs=[
            pl.BlockSpec((BB, D), lambda i: (i, 0)),
            pl.BlockSpec((BB, D), lambda i: (i, 0)),
            full(W1.shape),
            full((1, 128)),
            full(W2.shape),
            full((1, 64)),
            full(W3.shape),
            full((1, 32)),
            full((1, 32)),
            full((1, 1)),
        ],
        out_specs=pl.BlockSpec((BB, 1), lambda i: (i, 0)),
        out_shape=jax.ShapeDtypeStruct((B, 1), jnp.float32),
    )(u, v, W1, b1.reshape(1, -1), W2, b2.reshape(1, -1), W3, b3.reshape(1, -1),
      W4.reshape(1, -1), b4.reshape(1, -1))
    return out
